# Initial kernel scaffold; baseline (speedup 1.0000x reference)
import jax
import jax.numpy as jnp
from jax.experimental import pallas as pl


def kernel(x_agg_enc, W_enc_out_w, W_enc_out_b, codebook, W_dec_in_w, W_dec_in_b):
    raise NotImplementedError("write your pallas kernel here")


# speedup vs baseline: 1.6185x; 1.6185x over previous
import functools

import jax
import jax.numpy as jnp
from jax import lax
from jax.experimental import pallas as pl
from jax.experimental.pallas import tpu as pltpu
from jax.experimental.pallas import tpu_sc as plsc

LATENT = 32
HIDDEN = 512
EMBED = 1024
K = 512
B = 1024

_HIGHEST = jax.lax.Precision.HIGHEST


def _tc_body(a_ref, w_ref, b_ref, c_ref, v_ref, p_ref, ids_ref, loss_ref):
    l = pl.program_id(0)
    c = c_ref[...]
    xenc = (
        jnp.dot(a_ref[...], w_ref[...], preferred_element_type=jnp.float32)
        + b_ref[0]
    )
    x2 = jnp.sum(xenc * xenc, axis=1, keepdims=True)
    xw = jnp.dot(xenc, c, preferred_element_type=jnp.float32)
    w2 = jnp.sum(c * c, axis=0, keepdims=True)
    dist = x2 - 2.0 * xw + w2
    m = jnp.min(dist, axis=1, keepdims=True)
    kiota = lax.broadcasted_iota(jnp.int32, dist.shape, 1)
    idx = jnp.min(jnp.where(dist == m, kiota, K), axis=1)
    ids_ref[...] = (idx + l * K).reshape(1, 1, B)
    lane = lax.broadcasted_iota(jnp.int32, (1, 1, 128), 2)
    loss_ref[...] = jnp.where(lane == 0, jnp.sum(m), 0.0)
    p_ref[...] = lax.dot_general(
        c, v_ref[...], (((0,), (0,)), ((), ())),
        preferred_element_type=jnp.float32, precision=_HIGHEST)


def _tc_stage(A, W, benc, C, V):
    return pl.pallas_call(
        _tc_body,
        grid=(LATENT,),
        in_specs=[
            pl.BlockSpec((B, HIDDEN), lambda l: (0, 0)),
            pl.BlockSpec((HIDDEN, EMBED), lambda l: (0, l)),
            pl.BlockSpec((1, 1, EMBED), lambda l: (l, 0, 0)),
            pl.BlockSpec((EMBED, K), lambda l: (0, 0)),
            pl.BlockSpec((EMBED, HIDDEN), lambda l: (l, 0)),
        ],
        out_specs=[
            pl.BlockSpec((K, HIDDEN), lambda l: (l, 0)),
            pl.BlockSpec((1, 1, B), lambda l: (l, 0, 0)),
            pl.BlockSpec((1, 1, 128), lambda l: (l, 0, 0)),
        ],
        out_shape=[
            jax.ShapeDtypeStruct((LATENT * K, HIDDEN), jnp.float32),
            jax.ShapeDtypeStruct((LATENT, 1, B), jnp.int32),
            jax.ShapeDtypeStruct((LATENT, 1, 128), jnp.float32),
        ],
    )(A, W, benc, C, V)


_NW = 32
_NB = B // _NW


def _sc_body(p_hbm, ids_hbm, bdec_hbm, z_hbm, idsv, acc, rows, bias_v, sem):
    wid = lax.axis_index("s") * 2 + lax.axis_index("c")
    b0 = wid * _NB
    pltpu.sync_copy(ids_hbm, idsv)
    pltpu.sync_copy(bdec_hbm, bias_v)

    def _ids_slice(l):
        return idsv.at[pl.ds(pl.multiple_of(l * B + b0, 8), _NB)]

    def acc_one_l(l, _):
        pltpu.async_copy(p_hbm.at[_ids_slice(l)], rows, sem).wait()

        def row_loop(r, _):
            def col_loop(cc, _):
                sl = pl.ds(cc * 16, 16)
                acc[r, sl] += rows[r, sl]
                return 0
            return lax.fori_loop(0, HIDDEN // 16, col_loop, 0)
        lax.fori_loop(0, _NB, row_loop, 0)
        return 0

    pltpu.async_copy(p_hbm.at[_ids_slice(0)], rows, sem).wait()

    def init_row(r, _):
        def init_col(cc, _):
            sl = pl.ds(cc * 16, 16)
            acc[r, sl] = rows[r, sl] + bias_v[sl]
            return 0
        return lax.fori_loop(0, HIDDEN // 16, init_col, 0)
    lax.fori_loop(0, _NB, init_row, 0)

    lax.fori_loop(1, LATENT, acc_one_l, 0)
    pltpu.sync_copy(acc, z_hbm.at[pl.ds(b0, _NB)])


def _sc_stage(P, ids, bdec):
    mesh = plsc.VectorSubcoreMesh(core_axis_name="c", subcore_axis_name="s")
    kern = functools.partial(
        pl.kernel,
        mesh=mesh,
        out_type=jax.ShapeDtypeStruct((B, HIDDEN), jnp.float32),
        scratch_types=[
            pltpu.VMEM((LATENT * B,), jnp.int32),
            pltpu.VMEM((_NB, HIDDEN), jnp.float32),
            pltpu.VMEM((_NB, HIDDEN), jnp.float32),
            pltpu.VMEM((HIDDEN,), jnp.float32),
            pltpu.SemaphoreType.DMA,
        ],
    )(_sc_body)
    return kern(P, ids, bdec)


def kernel(x_agg_enc, W_enc_out_w, W_enc_out_b, codebook, W_dec_in_w, W_dec_in_b):
    benc = W_enc_out_b.reshape(LATENT, 1, EMBED)
    P, ids, lossparts = _tc_stage(x_agg_enc, W_enc_out_w, benc, codebook,
                                  W_dec_in_w)
    z = _sc_stage(P, ids.reshape(LATENT * B), W_dec_in_b)
    n = jnp.float32(B * LATENT * EMBED)
    total_loss = jnp.float32(1.25) * jnp.sum(lossparts[:, 0, 0]) / n
    return (z, total_loss)


# speedup vs baseline: 2.8076x; 1.7347x over previous
import functools

import jax
import jax.numpy as jnp
from jax import lax
from jax.experimental import pallas as pl
from jax.experimental.pallas import tpu as pltpu
from jax.experimental.pallas import tpu_sc as plsc

LATENT = 32
HIDDEN = 512
EMBED = 1024
K = 512
B = 1024

_HIGHEST = jax.lax.Precision.HIGHEST


def _tc_body(a_ref, w_ref, b_ref, c_ref, v_ref, p_ref, ids_ref, loss_ref):
    l = pl.program_id(0)
    c = c_ref[...]
    xenc = (
        jnp.dot(a_ref[...], w_ref[...], preferred_element_type=jnp.float32)
        + b_ref[0]
    )
    x2 = jnp.sum(xenc * xenc, axis=1, keepdims=True)
    xw = jnp.dot(xenc, c, preferred_element_type=jnp.float32)
    w2 = jnp.sum(c * c, axis=0, keepdims=True)
    dist = x2 - 2.0 * xw + w2
    m = jnp.min(dist, axis=1, keepdims=True)
    kiota = lax.broadcasted_iota(jnp.int32, dist.shape, 1)
    idx = jnp.min(jnp.where(dist == m, kiota, K), axis=1)
    ids_ref[...] = (idx + l * K).reshape(1, 1, B)
    lane = lax.broadcasted_iota(jnp.int32, (1, 1, 128), 2)
    loss_ref[...] = jnp.where(lane == 0, jnp.sum(m), 0.0)
    p_ref[...] = lax.dot_general(
        c, v_ref[...], (((0,), (0,)), ((), ())),
        preferred_element_type=jnp.float32)


def _tc_stage(A, W, benc, C, V):
    return pl.pallas_call(
        _tc_body,
        grid=(LATENT,),
        in_specs=[
            pl.BlockSpec((B, HIDDEN), lambda l: (0, 0)),
            pl.BlockSpec((HIDDEN, EMBED), lambda l: (0, l)),
            pl.BlockSpec((1, 1, EMBED), lambda l: (l, 0, 0)),
            pl.BlockSpec((EMBED, K), lambda l: (0, 0)),
            pl.BlockSpec((EMBED, HIDDEN), lambda l: (l, 0)),
        ],
        out_specs=[
            pl.BlockSpec((K, HIDDEN), lambda l: (l, 0)),
            pl.BlockSpec((1, 1, B), lambda l: (l, 0, 0)),
            pl.BlockSpec((1, 1, 128), lambda l: (l, 0, 0)),
        ],
        out_shape=[
            jax.ShapeDtypeStruct((LATENT * K, HIDDEN), jnp.float32),
            jax.ShapeDtypeStruct((LATENT, 1, B), jnp.int32),
            jax.ShapeDtypeStruct((LATENT, 1, 128), jnp.float32),
        ],
    )(A, W, benc, C, V)


_NW = 32
_NB = B // _NW


def _sc_body(p_hbm, ids_hbm, bdec_hbm, z_hbm, idsv, acc, rows0, rows1,
             bias_v, sem0, sem1):
    wid = lax.axis_index("s") * 2 + lax.axis_index("c")
    b0 = wid * _NB
    pltpu.sync_copy(ids_hbm, idsv)
    pltpu.sync_copy(bdec_hbm, bias_v)

    def _start(l, rbuf, sem):
        pltpu.async_copy(
            p_hbm.at[idsv.at[pl.ds(pl.multiple_of(l * B + b0, 8), _NB)]],
            rbuf, sem)

    def _wait(rbuf, sem):
        pltpu.make_async_copy(p_hbm.at[pl.ds(0, _NB)], rbuf, sem).wait()

    _start(0, rows0, sem0)
    _start(1, rows1, sem1)

    def init_row(r, _):
        for cc in range(HIDDEN // 16):
            sl = pl.ds(cc * 16, 16)
            acc[r, sl] = bias_v[sl]
        return 0
    lax.fori_loop(0, _NB, init_row, 0)

    def _accum(rbuf):
        def row_loop(r, _):
            for cc in range(HIDDEN // 16):
                sl = pl.ds(cc * 16, 16)
                acc[r, sl] += rbuf[r, sl]
            return 0
        lax.fori_loop(0, _NB, row_loop, 0)

    def pair_step(i, _):
        _wait(rows0, sem0)
        _accum(rows0)

        @pl.when(i < LATENT // 2 - 1)
        def _():
            _start(2 * i + 2, rows0, sem0)
        _wait(rows1, sem1)
        _accum(rows1)

        @pl.when(i < LATENT // 2 - 1)
        def _():
            _start(2 * i + 3, rows1, sem1)
        return 0

    lax.fori_loop(0, LATENT // 2, pair_step, 0)
    pltpu.sync_copy(acc, z_hbm.at[pl.ds(b0, _NB)])


def _sc_stage(P, ids, bdec):
    mesh = plsc.VectorSubcoreMesh(core_axis_name="c", subcore_axis_name="s")
    kern = functools.partial(
        pl.kernel,
        mesh=mesh,
        out_type=jax.ShapeDtypeStruct((B, HIDDEN), jnp.float32),
        scratch_types=[
            pltpu.VMEM((LATENT * B,), jnp.int32),
            pltpu.VMEM((_NB, HIDDEN), jnp.float32),
            pltpu.VMEM((_NB, HIDDEN), jnp.float32),
            pltpu.VMEM((_NB, HIDDEN), jnp.float32),
            pltpu.VMEM((HIDDEN,), jnp.float32),
            pltpu.SemaphoreType.DMA,
            pltpu.SemaphoreType.DMA,
        ],
    )(_sc_body)
    return kern(P, ids, bdec)


def kernel(x_agg_enc, W_enc_out_w, W_enc_out_b, codebook, W_dec_in_w, W_dec_in_b):
    benc = W_enc_out_b.reshape(LATENT, 1, EMBED)
    P, ids, lossparts = _tc_stage(x_agg_enc, W_enc_out_w, benc, codebook,
                                  W_dec_in_w)
    z = _sc_stage(P, ids.reshape(LATENT * B), W_dec_in_b)
    n = jnp.float32(B * LATENT * EMBED)
    total_loss = jnp.float32(1.25) * jnp.sum(lossparts[:, 0, 0]) / n
    return (z, total_loss)


# speedup vs baseline: 2.8781x; 1.0251x over previous
import functools

import jax
import jax.numpy as jnp
from jax import lax
from jax.experimental import pallas as pl
from jax.experimental.pallas import tpu as pltpu
from jax.experimental.pallas import tpu_sc as plsc

LATENT = 32
HIDDEN = 512
EMBED = 1024
K = 512
B = 1024

_HIGHEST = jax.lax.Precision.HIGHEST


def _tc_body(a_ref, w_ref, b_ref, c_ref, v_ref, p_ref, ids_ref, loss_ref):
    l = pl.program_id(0)
    c = c_ref[...]
    xenc = (
        jnp.dot(a_ref[...], w_ref[...], preferred_element_type=jnp.float32)
        + b_ref[0]
    )
    x2 = jnp.sum(xenc * xenc, axis=1, keepdims=True)
    xw = jnp.dot(xenc, c, preferred_element_type=jnp.float32)
    w2 = jnp.sum(c * c, axis=0, keepdims=True)
    dist = x2 - 2.0 * xw + w2
    m = jnp.min(dist, axis=1, keepdims=True)
    kiota = lax.broadcasted_iota(jnp.int32, dist.shape, 1)
    idx = jnp.min(jnp.where(dist == m, kiota, K), axis=1)
    ids_ref[...] = (idx + l * K).reshape(1, 1, B)
    lane = lax.broadcasted_iota(jnp.int32, (1, 1, 128), 2)
    loss_ref[...] = jnp.where(lane == 0, jnp.sum(m), 0.0)
    p_ref[...] = lax.dot_general(
        c, v_ref[...], (((0,), (0,)), ((), ())),
        preferred_element_type=jnp.float32)


def _tc_stage(A, W, benc, C, V):
    return pl.pallas_call(
        _tc_body,
        grid=(LATENT,),
        in_specs=[
            pl.BlockSpec((B, HIDDEN), lambda l: (0, 0)),
            pl.BlockSpec((HIDDEN, EMBED), lambda l: (0, l)),
            pl.BlockSpec((1, 1, EMBED), lambda l: (l, 0, 0)),
            pl.BlockSpec((EMBED, K), lambda l: (0, 0)),
            pl.BlockSpec((EMBED, HIDDEN), lambda l: (l, 0)),
        ],
        out_specs=[
            pl.BlockSpec((K, HIDDEN), lambda l: (l, 0)),
            pl.BlockSpec((1, 1, B), lambda l: (l, 0, 0)),
            pl.BlockSpec((1, 1, 128), lambda l: (l, 0, 0)),
        ],
        out_shape=[
            jax.ShapeDtypeStruct((LATENT * K, HIDDEN), jnp.float32),
            jax.ShapeDtypeStruct((LATENT, 1, B), jnp.int32),
            jax.ShapeDtypeStruct((LATENT, 1, 128), jnp.float32),
        ],
    )(A, W, benc, C, V)


_NW = 32
_NB = B // _NW


_G = 2


def _sc_body(p_hbm, ids_hbm, bdec_hbm, z_hbm, widsv, acc, rows0, rows1,
             bias_v, sem0, sem1):
    wid = lax.axis_index("s") * 2 + lax.axis_index("c")
    b0 = wid * _NB
    def stage_ids(l, _):
        pltpu.async_copy(
            ids_hbm.at[pl.ds(pl.multiple_of(l * B + b0, 8), _NB)],
            widsv.at[pl.ds(pl.multiple_of(l * _NB, 8), _NB)], sem0)
        return 0
    lax.fori_loop(0, LATENT, stage_ids, 0)
    pltpu.sync_copy(bdec_hbm, bias_v)
    pltpu.make_async_copy(ids_hbm.at[pl.ds(0, LATENT * _NB)], widsv,
                          sem0).wait()

    def _start(g, rbuf, sem):
        pltpu.async_copy(
            p_hbm.at[widsv.at[pl.ds(pl.multiple_of(g * _G * _NB, 8),
                                    _G * _NB)]],
            rbuf, sem)

    def _wait(rbuf, sem):
        pltpu.make_async_copy(p_hbm.at[pl.ds(0, _G * _NB)], rbuf, sem).wait()

    _start(0, rows0, sem0)
    _start(1, rows1, sem1)

    def init_row(r, _):
        for cc in range(HIDDEN // 16):
            sl = pl.ds(cc * 16, 16)
            acc[r, sl] = bias_v[sl]
        return 0
    lax.fori_loop(0, _NB, init_row, 0)

    def _accum(rbuf):
        def row_loop(r, _):
            for cc in range(HIDDEN // 16):
                sl = pl.ds(cc * 16, 16)
                acc[r, sl] += rbuf[r, sl] + rbuf[r + _NB, sl]
            return 0
        lax.fori_loop(0, _NB, row_loop, 0)

    _NG = LATENT // _G

    def pair_step(i, _):
        _wait(rows0, sem0)
        _accum(rows0)

        @pl.when(i < _NG // 2 - 1)
        def _():
            _start(2 * i + 2, rows0, sem0)
        _wait(rows1, sem1)
        _accum(rows1)

        @pl.when(i < _NG // 2 - 1)
        def _():
            _start(2 * i + 3, rows1, sem1)
        return 0

    lax.fori_loop(0, _NG // 2, pair_step, 0)
    pltpu.sync_copy(acc, z_hbm.at[pl.ds(b0, _NB)])


def _sc_stage(P, ids, bdec):
    mesh = plsc.VectorSubcoreMesh(core_axis_name="c", subcore_axis_name="s")
    kern = functools.partial(
        pl.kernel,
        mesh=mesh,
        out_type=jax.ShapeDtypeStruct((B, HIDDEN), jnp.float32),
        scratch_types=[
            pltpu.VMEM((LATENT * _NB,), jnp.int32),
            pltpu.VMEM((_NB, HIDDEN), jnp.float32),
            pltpu.VMEM((_G * _NB, HIDDEN), jnp.float32),
            pltpu.VMEM((_G * _NB, HIDDEN), jnp.float32),
            pltpu.VMEM((HIDDEN,), jnp.float32),
            pltpu.SemaphoreType.DMA,
            pltpu.SemaphoreType.DMA,
        ],
    )(_sc_body)
    return kern(P, ids, bdec)


def kernel(x_agg_enc, W_enc_out_w, W_enc_out_b, codebook, W_dec_in_w, W_dec_in_b):
    benc = W_enc_out_b.reshape(LATENT, 1, EMBED)
    P, ids, lossparts = _tc_stage(x_agg_enc, W_enc_out_w, benc, codebook,
                                  W_dec_in_w)
    z = _sc_stage(P, ids.reshape(LATENT * B), W_dec_in_b)
    n = jnp.float32(B * LATENT * EMBED)
    total_loss = jnp.float32(1.25) * jnp.sum(lossparts[:, 0, 0]) / n
    return (z, total_loss)


# speedup vs baseline: 2.9265x; 1.0168x over previous
import functools

import jax
import jax.numpy as jnp
from jax import lax
from jax.experimental import pallas as pl
from jax.experimental.pallas import tpu as pltpu
from jax.experimental.pallas import tpu_sc as plsc

LATENT = 32
HIDDEN = 512
EMBED = 1024
K = 512
B = 1024

_HIGHEST = jax.lax.Precision.HIGHEST


def _tc_body(a_ref, w_ref, b_ref, c_ref, v_ref, p_ref, ids_ref, loss_ref,
             xw_s, x2_s, w2_s):
    l = pl.program_id(0)
    c = c_ref[...]

    @pl.when(l == 0)
    def _():
        w2_s[...] = jnp.sum(c * c, axis=0, keepdims=True)

    dist = x2_s[...] - 2.0 * xw_s[...] + w2_s[...]
    m = jnp.min(dist, axis=1, keepdims=True)
    kiota = lax.broadcasted_iota(jnp.int32, dist.shape, 1)
    idx = jnp.min(jnp.where(dist == m, kiota, K), axis=1)
    ids_ref[...] = (idx + (l - 1) * K).reshape(1, 1, B)
    lane = lax.broadcasted_iota(jnp.int32, (1, 1, 128), 2)
    loss_ref[...] = jnp.where(lane == 0, jnp.sum(m), 0.0)

    xenc = (
        jnp.dot(a_ref[...], w_ref[...], preferred_element_type=jnp.float32)
        + b_ref[0]
    )
    x2_s[...] = jnp.sum(xenc * xenc, axis=1, keepdims=True)
    xw_s[...] = jnp.dot(xenc, c, preferred_element_type=jnp.float32)
    p_ref[...] = lax.dot_general(
        c, v_ref[...], (((0,), (0,)), ((), ())),
        preferred_element_type=jnp.float32)


def _tc_stage(A, W, benc, C, V):
    def _lo(l):
        return jnp.minimum(l, LATENT - 1)

    def _hi(l):
        return jnp.maximum(l - 1, 0)

    return pl.pallas_call(
        _tc_body,
        grid=(LATENT + 1,),
        in_specs=[
            pl.BlockSpec((B, HIDDEN), lambda l: (0, 0)),
            pl.BlockSpec((HIDDEN, EMBED), lambda l: (0, _lo(l))),
            pl.BlockSpec((1, 1, EMBED), lambda l: (_lo(l), 0, 0)),
            pl.BlockSpec((EMBED, K), lambda l: (0, 0)),
            pl.BlockSpec((EMBED, HIDDEN), lambda l: (_lo(l), 0)),
        ],
        out_specs=[
            pl.BlockSpec((K, HIDDEN), lambda l: (_lo(l), 0)),
            pl.BlockSpec((1, 1, B), lambda l: (_hi(l), 0, 0)),
            pl.BlockSpec((1, 1, 128), lambda l: (_hi(l), 0, 0)),
        ],
        out_shape=[
            jax.ShapeDtypeStruct((LATENT * K, HIDDEN), jnp.float32),
            jax.ShapeDtypeStruct((LATENT, 1, B), jnp.int32),
            jax.ShapeDtypeStruct((LATENT, 1, 128), jnp.float32),
        ],
        scratch_shapes=[
            pltpu.VMEM((B, K), jnp.float32),
            pltpu.VMEM((B, 1), jnp.float32),
            pltpu.VMEM((1, K), jnp.float32),
        ],
    )(A, W, benc, C, V)


_NW = 32
_NB = B // _NW


_G = 2


def _sc_body(p_hbm, ids_hbm, bdec_hbm, z_hbm, widsv, acc, rows0, rows1,
             bias_v, sem0, sem1):
    wid = lax.axis_index("s") * 2 + lax.axis_index("c")
    b0 = wid * _NB
    def stage_ids(l, _):
        pltpu.async_copy(
            ids_hbm.at[pl.ds(pl.multiple_of(l * B + b0, 8), _NB)],
            widsv.at[pl.ds(pl.multiple_of(l * _NB, 8), _NB)], sem0)
        return 0
    lax.fori_loop(0, LATENT, stage_ids, 0)
    pltpu.sync_copy(bdec_hbm, bias_v)
    pltpu.make_async_copy(ids_hbm.at[pl.ds(0, LATENT * _NB)], widsv,
                          sem0).wait()

    def _start(g, rbuf, sem):
        pltpu.async_copy(
            p_hbm.at[widsv.at[pl.ds(pl.multiple_of(g * _G * _NB, 8),
                                    _G * _NB)]],
            rbuf, sem)

    def _wait(rbuf, sem):
        pltpu.make_async_copy(p_hbm.at[pl.ds(0, _G * _NB)], rbuf, sem).wait()

    _start(0, rows0, sem0)
    _start(1, rows1, sem1)

    def init_row(r, _):
        for cc in range(HIDDEN // 16):
            sl = pl.ds(cc * 16, 16)
            acc[r, sl] = bias_v[sl]
        return 0
    lax.fori_loop(0, _NB, init_row, 0)

    def _accum(rbuf):
        def row_loop(r, _):
            for cc in range(HIDDEN // 16):
                sl = pl.ds(cc * 16, 16)
                acc[r, sl] += rbuf[r, sl] + rbuf[r + _NB, sl]
            return 0
        lax.fori_loop(0, _NB, row_loop, 0)

    _NG = LATENT // _G

    def pair_step(i, _):
        _wait(rows0, sem0)
        _accum(rows0)

        @pl.when(i < _NG // 2 - 1)
        def _():
            _start(2 * i + 2, rows0, sem0)
        _wait(rows1, sem1)
        _accum(rows1)

        @pl.when(i < _NG // 2 - 1)
        def _():
            _start(2 * i + 3, rows1, sem1)
        return 0

    lax.fori_loop(0, _NG // 2, pair_step, 0)
    pltpu.sync_copy(acc, z_hbm.at[pl.ds(b0, _NB)])


def _sc_stage(P, ids, bdec):
    mesh = plsc.VectorSubcoreMesh(core_axis_name="c", subcore_axis_name="s")
    kern = functools.partial(
        pl.kernel,
        mesh=mesh,
        out_type=jax.ShapeDtypeStruct((B, HIDDEN), jnp.float32),
        scratch_types=[
            pltpu.VMEM((LATENT * _NB,), jnp.int32),
            pltpu.VMEM((_NB, HIDDEN), jnp.float32),
            pltpu.VMEM((_G * _NB, HIDDEN), jnp.float32),
            pltpu.VMEM((_G * _NB, HIDDEN), jnp.float32),
            pltpu.VMEM((HIDDEN,), jnp.float32),
            pltpu.SemaphoreType.DMA,
            pltpu.SemaphoreType.DMA,
        ],
    )(_sc_body)
    return kern(P, ids, bdec)


def kernel(x_agg_enc, W_enc_out_w, W_enc_out_b, codebook, W_dec_in_w, W_dec_in_b):
    benc = W_enc_out_b.reshape(LATENT, 1, EMBED)
    P, ids, lossparts = _tc_stage(x_agg_enc, W_enc_out_w, benc, codebook,
                                  W_dec_in_w)
    z = _sc_stage(P, ids.reshape(LATENT * B), W_dec_in_b)
    n = jnp.float32(B * LATENT * EMBED)
    total_loss = jnp.float32(1.25) * jnp.sum(lossparts[:, 0, 0]) / n
    return (z, total_loss)


# speedup vs baseline: 3.1007x; 1.0595x over previous
import functools

import jax
import jax.numpy as jnp
from jax import lax
from jax.experimental import pallas as pl
from jax.experimental.pallas import tpu as pltpu
from jax.experimental.pallas import tpu_sc as plsc

LATENT = 32
HIDDEN = 512
EMBED = 1024
K = 512
B = 1024

_HIGHEST = jax.lax.Precision.HIGHEST


def _tc_body(a_ref, w_ref, b_ref, c_ref, v_ref, p_ref, ids_ref, loss_ref,
             xw_s, x2_s, w2_s):
    l = pl.program_id(0)
    c = c_ref[...]

    @pl.when(l == 0)
    def _():
        w2_s[...] = jnp.sum(c * c, axis=0, keepdims=True)

    dist = x2_s[...] - 2.0 * xw_s[...] + w2_s[...]
    m = jnp.min(dist, axis=1, keepdims=True)
    kiota = lax.broadcasted_iota(jnp.int32, dist.shape, 1)
    idx = jnp.min(jnp.where(dist == m, kiota, K), axis=1)
    ids_ref[...] = (idx + (l - 1) * K).reshape(1, 1, B)
    lane = lax.broadcasted_iota(jnp.int32, (1, 1, 128), 2)
    loss_ref[...] = jnp.where(lane == 0, jnp.sum(m), 0.0)

    xenc = (
        jnp.dot(a_ref[...], w_ref[...], preferred_element_type=jnp.float32)
        + b_ref[0]
    )
    x2_s[...] = jnp.sum(xenc * xenc, axis=1, keepdims=True)
    xw_s[...] = jnp.dot(xenc, c, preferred_element_type=jnp.float32)
    p_ref[...] = lax.dot_general(
        c, v_ref[...], (((0,), (0,)), ((), ())),
        preferred_element_type=jnp.float32)


def _tc_stage(A, W, benc, C, V):
    def _lo(l):
        return jnp.minimum(l, LATENT - 1)

    def _hi(l):
        return jnp.maximum(l - 1, 0)

    return pl.pallas_call(
        _tc_body,
        grid=(LATENT + 1,),
        in_specs=[
            pl.BlockSpec((B, HIDDEN), lambda l: (0, 0)),
            pl.BlockSpec((HIDDEN, EMBED), lambda l: (0, _lo(l))),
            pl.BlockSpec((1, 1, EMBED), lambda l: (_lo(l), 0, 0)),
            pl.BlockSpec((EMBED, K), lambda l: (0, 0)),
            pl.BlockSpec((EMBED, HIDDEN), lambda l: (_lo(l), 0)),
        ],
        out_specs=[
            pl.BlockSpec((K, HIDDEN), lambda l: (_lo(l), 0)),
            pl.BlockSpec((1, 1, B), lambda l: (_hi(l), 0, 0)),
            pl.BlockSpec((1, 1, 128), lambda l: (_hi(l), 0, 0)),
        ],
        out_shape=[
            jax.ShapeDtypeStruct((LATENT * K, HIDDEN), jnp.float32),
            jax.ShapeDtypeStruct((LATENT, 1, B), jnp.int32),
            jax.ShapeDtypeStruct((LATENT, 1, 128), jnp.float32),
        ],
        scratch_shapes=[
            pltpu.VMEM((B, K), jnp.float32),
            pltpu.VMEM((B, 1), jnp.float32),
            pltpu.VMEM((1, K), jnp.float32),
        ],
    )(A, W, benc, C, V)


_NW = 32
_NB = B // _NW


_G = 2


def _sc_body(p_hbm, ids_hbm, bdec_hbm, z_hbm, widsv, acc, rows0, rows1,
             bias_v, sem0, sem1):
    wid = lax.axis_index("s") * 2 + lax.axis_index("c")
    b0 = wid * _NB
    def stage_ids(l, _):
        pltpu.async_copy(
            ids_hbm.at[pl.ds(pl.multiple_of(l * B + b0, 8), _NB)],
            widsv.at[pl.ds(pl.multiple_of(l * _NB, 8), _NB)], sem0)
        return 0
    lax.fori_loop(0, LATENT, stage_ids, 0)
    pltpu.sync_copy(bdec_hbm, bias_v)
    pltpu.make_async_copy(ids_hbm.at[pl.ds(0, LATENT * _NB)], widsv,
                          sem0).wait()

    def _start(g, rbuf, sem):
        pltpu.async_copy(
            p_hbm.at[widsv.at[pl.ds(pl.multiple_of(g * _G * _NB, 8),
                                    _G * _NB)]],
            rbuf, sem)

    def _wait(rbuf, sem):
        pltpu.make_async_copy(p_hbm.at[pl.ds(0, _G * _NB)], rbuf, sem).wait()

    _start(0, rows0, sem0)
    _start(1, rows1, sem1)

    @plsc.parallel_loop(0, _NB, step=1, carry=jnp.int32(0))
    def init_row(r, j):
        for cc in range(HIDDEN // 16):
            sl = pl.ds(cc * 16, 16)
            acc[r, sl] = bias_v[sl]
        return j

    def _accum(rbuf):
        @plsc.parallel_loop(0, _NB, step=1, carry=jnp.int32(0))
        def row_loop(r, j):
            for cc in range(HIDDEN // 16):
                sl = pl.ds(cc * 16, 16)
                acc[r, sl] += rbuf[r, sl] + rbuf[r + _NB, sl]
            return j

    _NG = LATENT // _G

    def pair_step(i, _):
        _wait(rows0, sem0)
        _accum(rows0)

        @pl.when(i < _NG // 2 - 1)
        def _():
            _start(2 * i + 2, rows0, sem0)
        _wait(rows1, sem1)
        _accum(rows1)

        @pl.when(i < _NG // 2 - 1)
        def _():
            _start(2 * i + 3, rows1, sem1)
        return 0

    lax.fori_loop(0, _NG // 2, pair_step, 0)
    pltpu.sync_copy(acc, z_hbm.at[pl.ds(b0, _NB)])


def _sc_stage(P, ids, bdec):
    mesh = plsc.VectorSubcoreMesh(core_axis_name="c", subcore_axis_name="s")
    kern = functools.partial(
        pl.kernel,
        mesh=mesh,
        out_type=jax.ShapeDtypeStruct((B, HIDDEN), jnp.float32),
        scratch_types=[
            pltpu.VMEM((LATENT * _NB,), jnp.int32),
            pltpu.VMEM((_NB, HIDDEN), jnp.float32),
            pltpu.VMEM((_G * _NB, HIDDEN), jnp.float32),
            pltpu.VMEM((_G * _NB, HIDDEN), jnp.float32),
            pltpu.VMEM((HIDDEN,), jnp.float32),
            pltpu.SemaphoreType.DMA,
            pltpu.SemaphoreType.DMA,
        ],
    )(_sc_body)
    return kern(P, ids, bdec)


def kernel(x_agg_enc, W_enc_out_w, W_enc_out_b, codebook, W_dec_in_w, W_dec_in_b):
    benc = W_enc_out_b.reshape(LATENT, 1, EMBED)
    P, ids, lossparts = _tc_stage(x_agg_enc, W_enc_out_w, benc, codebook,
                                  W_dec_in_w)
    z = _sc_stage(P, ids.reshape(LATENT * B), W_dec_in_b)
    n = jnp.float32(B * LATENT * EMBED)
    total_loss = jnp.float32(1.25) * jnp.sum(lossparts[:, 0, 0]) / n
    return (z, total_loss)
